# Initial kernel scaffold; baseline (speedup 1.0000x reference)
import jax
import jax.numpy as jnp
from jax.experimental import pallas as pl


def kernel(x, edge_index, edge_attr, W_self, W_nbr, W_edge, b_gnn, W1, b1, W2, b2, W3, b3):
    raise NotImplementedError("write your pallas kernel here")


# speedup vs baseline: 4.4502x; 4.4502x over previous
import functools

import jax
import jax.numpy as jnp
from jax import lax
from jax.experimental import pallas as pl
from jax.experimental.pallas import tpu as pltpu
from jax.experimental.pallas import tpu_sc as plsc

N = 10000
E = 320000
D = 128
DE = 16
H = 128
OUTC = 8

NC = 2
NS = 16
NW = NC * NS
CHUNK = 80
CPW = E // (NW * CHUNK)
GRP = 5
CPG = CPW // GRP
N_PAD = 10240
RPT = N_PAD // NS


def _sc_segsum_x(src3d, dst3d, x, zx):
    mesh = plsc.VectorSubcoreMesh(core_axis_name="c", subcore_axis_name="s")

    @functools.partial(
        pl.kernel,
        mesh=mesh,
        out_type=jax.ShapeDtypeStruct((NC, N_PAD, D), jnp.float32),
        scratch_types=[
            pltpu.VMEM((CPG, CHUNK), jnp.int32),
            pltpu.VMEM((CPG, CHUNK), jnp.int32),
            pltpu.VMEM((CHUNK, D), jnp.float32),
            pltpu.VMEM_SHARED((N_PAD, D), jnp.float32),
            pltpu.SemaphoreType.DMA,
        ],
    )
    def k(src_hbm, dst_hbm, x_hbm, zx_hbm, outx, srcb, dstb, rowsb, accx, sem):
        c = lax.axis_index("c")
        s = lax.axis_index("s")
        wid = s * NC + c
        pltpu.sync_copy(zx_hbm, accx.at[pl.ds(s * RPT, RPT)])
        plsc.subcore_barrier()

        for grp in range(GRP):
            pltpu.sync_copy(src_hbm.at[wid, grp], srcb)
            pltpu.sync_copy(dst_hbm.at[wid, grp], dstb)

            def body(j, carry):
                pltpu.async_copy(x_hbm.at[srcb.at[j]], rowsb, sem).wait()
                pltpu.sync_copy(rowsb, accx.at[dstb.at[j]], add=True)
                return carry

            lax.fori_loop(0, CPG, body, 0)
        plsc.subcore_barrier()
        pltpu.sync_copy(accx.at[pl.ds(s * RPT, RPT)], outx.at[c, pl.ds(s * RPT, RPT)])

    return k(src3d, dst3d, x, zx)


def _sc_segsum_ea(dst3d, edge_attr, ze):
    mesh = plsc.VectorSubcoreMesh(core_axis_name="c", subcore_axis_name="s")

    @functools.partial(
        pl.kernel,
        mesh=mesh,
        out_type=jax.ShapeDtypeStruct((NC, N_PAD, DE), jnp.float32),
        scratch_types=[
            pltpu.VMEM((CPG, CHUNK), jnp.int32),
            pltpu.VMEM((CHUNK, DE), jnp.float32),
            pltpu.VMEM_SHARED((N_PAD, DE), jnp.float32),
            pltpu.SemaphoreType.DMA,
        ],
    )
    def k(dst_hbm, ea_hbm, ze_hbm, oute, dstb, eab, acce, sem):
        c = lax.axis_index("c")
        s = lax.axis_index("s")
        wid = s * NC + c
        pltpu.sync_copy(ze_hbm, acce.at[pl.ds(s * RPT, RPT)])
        plsc.subcore_barrier()

        for grp in range(GRP):
            pltpu.sync_copy(dst_hbm.at[wid, grp], dstb)

            def body(j, carry):
                g = (wid * GRP + grp) * CPG + j
                pltpu.sync_copy(ea_hbm.at[pl.ds(g * CHUNK, CHUNK)], eab)
                pltpu.sync_copy(eab, acce.at[dstb.at[j]], add=True)
                return carry

            lax.fori_loop(0, CPG, body, 0)
        plsc.subcore_barrier()
        pltpu.sync_copy(acce.at[pl.ds(s * RPT, RPT)], oute.at[c, pl.ds(s * RPT, RPT)])

    return k(dst3d, edge_attr, ze)


def _tc_head(x, px, pe, W_self, W_nbr, W_edge, b_gnn, W1, b1, W2, b2, W3, b3):
    BR = 1000

    def body(x_r, px_r, pe_r, ws_r, wn_r, we_r, bg_r, w1_r, b1_r, w2_r,
             b2_r, w3_r, b3_r, o_r):
        agg = px_r[0] + px_r[1]
        ae = pe_r[0] + pe_r[1]
        h = jnp.dot(x_r[...], ws_r[...], preferred_element_type=jnp.float32)
        h = h + jnp.dot(agg, wn_r[...], preferred_element_type=jnp.float32)
        h = h + jnp.dot(ae, we_r[...], preferred_element_type=jnp.float32)
        h = jnp.maximum(h + bg_r[...], 0.0)
        h = jnp.maximum(
            jnp.dot(h, w1_r[...], preferred_element_type=jnp.float32) + b1_r[...], 0.0)
        h = jnp.maximum(
            jnp.dot(h, w2_r[...], preferred_element_type=jnp.float32) + b2_r[...], 0.0)
        o_r[...] = jnp.dot(h, w3_r[...], preferred_element_type=jnp.float32) + b3_r[...]

    full = lambda shape: pl.BlockSpec(shape, lambda i: tuple(0 for _ in shape))
    return pl.pallas_call(
        body,
        grid=(N // BR,),
        in_specs=[
            pl.BlockSpec((BR, D), lambda i: (i, 0)),
            pl.BlockSpec((NC, BR, D), lambda i: (0, i, 0)),
            pl.BlockSpec((NC, BR, DE), lambda i: (0, i, 0)),
            full((D, H)),
            full((D, H)),
            full((DE, H)),
            full((1, H)),
            full((H, 32)),
            full((1, 32)),
            full((32, 16)),
            full((1, 16)),
            full((16, OUTC)),
            full((1, OUTC)),
        ],
        out_specs=pl.BlockSpec((BR, OUTC), lambda i: (i, 0)),
        out_shape=jax.ShapeDtypeStruct((N, OUTC), jnp.float32),
    )(x, px, pe, W_self, W_nbr, W_edge, b_gnn, W1, b1, W2, b2, W3, b3)


def kernel(x, edge_index, edge_attr, W_self, W_nbr, W_edge, b_gnn, W1, b1,
           W2, b2, W3, b3):
    src3d = edge_index[0].reshape(NW, GRP, CPG, CHUNK)
    dst3d = edge_index[1].reshape(NW, GRP, CPG, CHUNK)
    zx = jnp.zeros((RPT, D), jnp.float32)
    ze = jnp.zeros((RPT, DE), jnp.float32)
    px = _sc_segsum_x(src3d, dst3d, x, zx)
    pe = _sc_segsum_ea(dst3d, edge_attr, ze)
    return _tc_head(
        x, px, pe, W_self, W_nbr, W_edge,
        b_gnn.reshape(1, H), W1, b1.reshape(1, 32), W2, b2.reshape(1, 16),
        W3, b3.reshape(1, OUTC))
